# baseline (speedup 1.0000x reference)
import functools

import jax
import jax.numpy as jnp
from jax import lax
from jax.experimental import pallas as pl
from jax.experimental.pallas import tpu as pltpu
from jax.experimental.pallas import tpu_sc as plsc

B = 16384
D = 64
MARGIN = 1.0
PAD_IDX = 0

NC = 2
NS = 16
L = 16
NW = NC * NS
BPW = B // NW
IDX_CHUNK = 128
NP = 4
RPP = BPW // NP
DC = D // L


def _tl_kernel(x_hbm, tgt_hbm, nid_hbm, emb2_hbm, out_hbm,
               tgt_v, nid_v, jt_v, jn_v, x_v, pos_v, neg_v, out_v, sem):
    wid = lax.axis_index("s") * NC + lax.axis_index("c")
    base = wid * BPW

    pltpu.sync_copy(tgt_hbm.at[pl.ds(base, BPW)], tgt_v)
    pltpu.sync_copy(nid_hbm.at[pl.ds(base, BPW)], nid_v)
    x_copy = pltpu.async_copy(x_hbm.at[pl.ds(base, BPW)], x_v, sem)

    row_iota = lax.iota(jnp.int32, L)
    perms = [row_iota ^ k for k in (8, 4, 2, 1)]
    dnums = lax.GatherDimensionNumbers(
        offset_dims=(), collapsed_slice_dims=(0,), start_index_map=(0,))

    def lane_perm(v, idx):
        return lax.gather(v, idx[:, None], dnums, slice_sizes=(1,),
                          mode=lax.GatherScatterMode.PROMISE_IN_BOUNDS)

    def hsum(v):
        for p in perms:
            v = v + lane_perm(v, p)
        return v

    for k in range(NP):
        for ch in range(RPP // L):
            sg = pl.ds(k * RPP + ch * L, L)
            sl = pl.ds(ch * L, L)
            jt_v[sl] = lax.shift_right_logical(tgt_v[sg], 1)
            jn_v[sl] = lax.shift_right_logical(nid_v[sg], 1)
        gathers = []
        for j in range(RPP // IDX_CHUNK):
            s = pl.ds(j * IDX_CHUNK, IDX_CHUNK)
            gathers.append(pltpu.async_copy(emb2_hbm.at[jt_v.at[s]], pos_v.at[s], sem))
            gathers.append(pltpu.async_copy(emb2_hbm.at[jn_v.at[s]], neg_v.at[s], sem))
        if k == 0:
            x_copy.wait()
        for c in gathers:
            c.wait()

        def group_body(g, carry):
            rb = g * L
            tvec = tgt_v[pl.ds(k * RPP + rb, L)]
            nvec = nid_v[pl.ds(k * RPP + rb, L)]
            tpar = (tvec & 1) * D
            npar = (nvec & 1) * D
            ovec = jnp.zeros((L,), jnp.float32)
            for i in range(L):
                r = rb + i
                gr = k * RPP + r
                tb = tpar[i]
                nb = npar[i]
                acc = jnp.zeros((L,), jnp.float32)
                for c in range(DC):
                    xc = x_v[gr, pl.ds(c * L, L)]
                    pc = pos_v[r, pl.ds(tb + c * L, L)]
                    nc = neg_v[r, pl.ds(nb + c * L, L)]
                    acc = acc + (pc * pc - nc * nc - (xc + xc) * (pc - nc))
                ovec = jnp.where(row_iota == i, hsum(acc), ovec)
            loss = jnp.maximum(ovec + MARGIN, 0.0)
            out_v[pl.ds(k * RPP + rb, L)] = jnp.where(tvec == PAD_IDX, 0.0, loss)
            return carry

        lax.fori_loop(0, RPP // L, group_body, 0)

    pltpu.sync_copy(out_v, out_hbm.at[pl.ds(base, BPW)])


@functools.partial(
    pl.kernel,
    mesh=plsc.VectorSubcoreMesh(core_axis_name="c", subcore_axis_name="s"),
    out_type=jax.ShapeDtypeStruct((B,), jnp.float32),
    scratch_types=[
        pltpu.VMEM((BPW,), jnp.int32),
        pltpu.VMEM((BPW,), jnp.int32),
        pltpu.VMEM((RPP,), jnp.int32),
        pltpu.VMEM((RPP,), jnp.int32),
        pltpu.VMEM((BPW, D), jnp.float32),
        pltpu.VMEM((RPP, 2 * D), jnp.float32),
        pltpu.VMEM((RPP, 2 * D), jnp.float32),
        pltpu.VMEM((BPW,), jnp.float32),
        pltpu.SemaphoreType.DMA,
    ],
)
def _tl_call(x_hbm, tgt_hbm, nid_hbm, emb2_hbm, out_hbm,
             tgt_v, nid_v, jt_v, jn_v, x_v, pos_v, neg_v, out_v, sem):
    _tl_kernel(x_hbm, tgt_hbm, nid_hbm, emb2_hbm, out_hbm,
               tgt_v, nid_v, jt_v, jn_v, x_v, pos_v, neg_v, out_v, sem)


def kernel(x, targets, emb, neg_ids):
    emb2 = emb.reshape(emb.shape[0] // 2, 2 * D)
    return _tl_call(x, targets, neg_ids, emb2)


# speedup vs baseline: 1.6033x; 1.6033x over previous
import functools

import jax
import jax.numpy as jnp
from jax import lax
from jax.experimental import pallas as pl
from jax.experimental.pallas import tpu as pltpu
from jax.experimental.pallas import tpu_sc as plsc

B = 16384
D = 64
MARGIN = 1.0
PAD_IDX = 0

NC = 2
NS = 16
L = 16
NW = NC * NS
BPW = B // NW
GROUPS = BPW // L
DC = D // L


def _tl_kernel(x_hbm, tgt_hbm, nid_hbm, emb_hbm, out_hbm,
               tgt_v, nid_v, x_v, pbuf, nbuf, out_v, sem):
    wid = lax.axis_index("s") * NC + lax.axis_index("c")
    base = wid * BPW

    pltpu.sync_copy(tgt_hbm.at[pl.ds(base, BPW)], tgt_v)
    pltpu.sync_copy(nid_hbm.at[pl.ds(base, BPW)], nid_v)
    x_copy = pltpu.async_copy(x_hbm.at[pl.ds(base, BPW)], x_v, sem)

    row_iota = lax.iota(jnp.int32, L)
    perms = [row_iota ^ k for k in (8, 4, 2, 1)]
    dnums = lax.GatherDimensionNumbers(
        offset_dims=(), collapsed_slice_dims=(0,), start_index_map=(0,))

    def lane_perm(v, idx):
        return lax.gather(v, idx[:, None], dnums, slice_sizes=(1,),
                          mode=lax.GatherScatterMode.PROMISE_IN_BOUNDS)

    def hsum(v):
        for p in perms:
            v = v + lane_perm(v, p)
        return v

    x_copy.wait()

    def group_body(g, carry):
        rb = g * L
        tvec = tgt_v[pl.ds(rb, L)]
        nvec = nid_v[pl.ds(rb, L)]
        cps = []
        for i in range(L):
            cps.append(pltpu.async_copy(
                emb_hbm.at[pl.ds(tvec[i], 1)], pbuf.at[pl.ds(i, 1)], sem))
            cps.append(pltpu.async_copy(
                emb_hbm.at[pl.ds(nvec[i], 1)], nbuf.at[pl.ds(i, 1)], sem))
        for cp in cps:
            cp.wait()
        ovec = jnp.zeros((L,), jnp.float32)
        for i in range(L):
            acc = jnp.zeros((L,), jnp.float32)
            for c in range(DC):
                cs = pl.ds(c * L, L)
                xc = x_v[rb + i, cs]
                pc = pbuf[i, cs]
                nc = nbuf[i, cs]
                acc = acc + (pc * pc - nc * nc - (xc + xc) * (pc - nc))
            ovec = jnp.where(row_iota == i, hsum(acc), ovec)
        loss = jnp.maximum(ovec + MARGIN, 0.0)
        out_v[pl.ds(rb, L)] = jnp.where(tvec == PAD_IDX, 0.0, loss)
        return carry

    lax.fori_loop(0, GROUPS, group_body, 0)

    pltpu.sync_copy(out_v, out_hbm.at[pl.ds(base, BPW)])


@functools.partial(
    pl.kernel,
    mesh=plsc.VectorSubcoreMesh(core_axis_name="c", subcore_axis_name="s"),
    out_type=jax.ShapeDtypeStruct((B,), jnp.float32),
    scratch_types=[
        pltpu.VMEM((BPW,), jnp.int32),
        pltpu.VMEM((BPW,), jnp.int32),
        pltpu.VMEM((BPW, D), jnp.float32),
        pltpu.VMEM((L, D), jnp.float32),
        pltpu.VMEM((L, D), jnp.float32),
        pltpu.VMEM((BPW,), jnp.float32),
        pltpu.SemaphoreType.DMA,
    ],
)
def _tl_call(x_hbm, tgt_hbm, nid_hbm, emb_hbm, out_hbm,
             tgt_v, nid_v, x_v, pbuf, nbuf, out_v, sem):
    _tl_kernel(x_hbm, tgt_hbm, nid_hbm, emb_hbm, out_hbm,
               tgt_v, nid_v, x_v, pbuf, nbuf, out_v, sem)


def kernel(x, targets, emb, neg_ids):
    return _tl_call(x, targets, neg_ids, emb)


# speedup vs baseline: 1.6679x; 1.0403x over previous
import functools

import jax
import jax.numpy as jnp
from jax import lax
from jax.experimental import pallas as pl
from jax.experimental.pallas import tpu as pltpu
from jax.experimental.pallas import tpu_sc as plsc

B = 16384
D = 64
MARGIN = 1.0
PAD_IDX = 0

NC = 2
NS = 16
L = 16
NW = NC * NS
BPW = B // NW
GROUPS = BPW // L
DC = D // L
NP = 2
RPP = BPW // NP


def _tl_kernel(x_hbm, tgt_hbm, nid_hbm, emb_hbm, out_hbm,
               tgt_v, nid_v, x_v, pbuf, nbuf, out_v, sem):
    wid = lax.axis_index("s") * NC + lax.axis_index("c")
    base = wid * BPW

    pltpu.sync_copy(tgt_hbm.at[pl.ds(base, BPW)], tgt_v)
    pltpu.sync_copy(nid_hbm.at[pl.ds(base, BPW)], nid_v)

    row_iota = lax.iota(jnp.int32, L)
    perms = [row_iota ^ k for k in (8, 4, 2, 1)]
    dnums = lax.GatherDimensionNumbers(
        offset_dims=(), collapsed_slice_dims=(0,), start_index_map=(0,))

    def lane_perm(v, idx):
        return lax.gather(v, idx[:, None], dnums, slice_sizes=(1,),
                          mode=lax.GatherScatterMode.PROMISE_IN_BOUNDS)

    def hsum(v):
        for p in perms:
            v = v + lane_perm(v, p)
        return v

    GPP = GROUPS // NP

    for ph in range(NP):
        pb = ph * GPP * L
        pltpu.sync_copy(x_hbm.at[pl.ds(base + pb, RPP)], x_v)

        def issue_body(g, carry):
            rb = pb + g * L
            tvec = tgt_v[pl.ds(rb, L)]
            nvec = nid_v[pl.ds(rb, L)]
            for i in range(L):
                pltpu.async_copy(
                    emb_hbm.at[pl.ds(tvec[i], 1)], pbuf.at[pl.ds(g * L + i, 1)], sem)
                pltpu.async_copy(
                    emb_hbm.at[pl.ds(nvec[i], 1)], nbuf.at[pl.ds(g * L + i, 1)], sem)
            return carry

        lax.fori_loop(0, GPP, issue_body, 0)

        def drain_body(g, carry):
            for _ in range(2 * L):
                pltpu.make_async_copy(
                    emb_hbm.at[pl.ds(0, 1)], pbuf.at[pl.ds(0, 1)], sem).wait()
            return carry

        lax.fori_loop(0, GPP, drain_body, 0)

        def group_body(g, carry):
            rb = pb + g * L
            tvec = tgt_v[pl.ds(rb, L)]
            ovec = jnp.zeros((L,), jnp.float32)
            for i in range(L):
                acc = jnp.zeros((L,), jnp.float32)
                for c in range(DC):
                    cs = pl.ds(c * L, L)
                    xc = x_v[g * L + i, cs]
                    pc = pbuf[g * L + i, cs]
                    nc = nbuf[g * L + i, cs]
                    acc = acc + (pc * pc - nc * nc - (xc + xc) * (pc - nc))
                ovec = jnp.where(row_iota == i, hsum(acc), ovec)
            loss = jnp.maximum(ovec + MARGIN, 0.0)
            out_v[pl.ds(rb, L)] = jnp.where(tvec == PAD_IDX, 0.0, loss)
            return carry

        lax.fori_loop(0, GPP, group_body, 0)

    pltpu.sync_copy(out_v, out_hbm.at[pl.ds(base, BPW)])


@functools.partial(
    pl.kernel,
    mesh=plsc.VectorSubcoreMesh(core_axis_name="c", subcore_axis_name="s"),
    out_type=jax.ShapeDtypeStruct((B,), jnp.float32),
    scratch_types=[
        pltpu.VMEM((BPW,), jnp.int32),
        pltpu.VMEM((BPW,), jnp.int32),
        pltpu.VMEM((RPP, D), jnp.float32),
        pltpu.VMEM((RPP, D), jnp.float32),
        pltpu.VMEM((RPP, D), jnp.float32),
        pltpu.VMEM((BPW,), jnp.float32),
        pltpu.SemaphoreType.DMA,
    ],
)
def _tl_call(x_hbm, tgt_hbm, nid_hbm, emb_hbm, out_hbm,
             tgt_v, nid_v, x_v, pbuf, nbuf, out_v, sem):
    _tl_kernel(x_hbm, tgt_hbm, nid_hbm, emb_hbm, out_hbm,
               tgt_v, nid_v, x_v, pbuf, nbuf, out_v, sem)


def kernel(x, targets, emb, neg_ids):
    return _tl_call(x, targets, neg_ids, emb)
